# baseline (speedup 1.0000x reference)
import functools

import jax
import jax.numpy as jnp
from jax import lax
from jax.experimental import pallas as pl
from jax.experimental.pallas import tpu as pltpu
from jax.experimental.pallas import tpu_sc as plsc

D = 18432
B = 4096
K = 2
NBUF = 2
NCORES = 2
NSUB = 16
NW = NCORES * NSUB
NCH = B // K
CH_PER_W = NCH // NW


def _sc_gather(idx2d, table):
    mesh = plsc.VectorSubcoreMesh(core_axis_name="c", subcore_axis_name="s")

    @functools.partial(
        pl.kernel,
        out_type=jax.ShapeDtypeStruct((NCH, K, D), jnp.float32),
        mesh=mesh,
        scratch_types=[
            pltpu.VMEM((CH_PER_W, K), jnp.int32),
            pltpu.VMEM((NBUF, K, D), jnp.float32),
            pltpu.SemaphoreType.DMA,
            pltpu.SemaphoreType.DMA,
            pltpu.SemaphoreType.DMA,
            pltpu.SemaphoreType.DMA,
        ],
    )
    def k(idx_hbm, table_hbm, out_hbm, idx_v, buf, gsem0, gsem1, ssem0, ssem1):
        gsems = (gsem0, gsem1)
        ssems = (ssem0, ssem1)
        wid = lax.axis_index("s") * NCORES + lax.axis_index("c")
        base = wid * CH_PER_W

        pltpu.sync_copy(idx_hbm.at[pl.ds(base, CH_PER_W)], idx_v)

        def gather(slot, c_local):
            return pltpu.make_async_copy(
                table_hbm.at[idx_v.at[c_local]], buf.at[slot], gsems[slot])

        def scatter(slot, c_local):
            return pltpu.make_async_copy(
                buf.at[slot], out_hbm.at[base + c_local], ssems[slot])

        for b in range(NBUF):
            gather(b, b).start()

        def step(g, carry):
            for b in range(NBUF):
                c = g * NBUF + b
                gather(b, c).wait()
                scatter(b, c).start()
                scatter(b, c).wait()

                @pl.when(c + NBUF < CH_PER_W)
                def _():
                    gather(b, c + NBUF).start()

            return carry

        lax.fori_loop(0, CH_PER_W // NBUF, step, 0)

    return k(idx2d, table)


def kernel(prefix, embedding):
    idx2d = prefix.reshape(NCH, K)
    out = _sc_gather(idx2d, embedding)
    return out.reshape(prefix.shape[0], prefix.shape[1], D)


# speedup vs baseline: 4.4920x; 4.4920x over previous
import functools

import jax
import jax.numpy as jnp
from jax import lax
from jax.experimental import pallas as pl
from jax.experimental.pallas import tpu as pltpu
from jax.experimental.pallas import tpu_sc as plsc

D = 18432
B = 4096
K = 2
NBUF = 2
NCORES = 2
NSUB = 16
NW = NCORES * NSUB
NCH = B // K
CH_PER_W = NCH // NW


BATCH = 32
SEQ = 128


def _sc_gather(idx2d, table):
    mesh = plsc.VectorSubcoreMesh(core_axis_name="c", subcore_axis_name="s")

    @functools.partial(
        pl.kernel,
        out_type=jax.ShapeDtypeStruct((BATCH, SEQ, D), jnp.float32),
        mesh=mesh,
        scratch_types=[
            pltpu.VMEM((CH_PER_W, K), jnp.int32),
            pltpu.VMEM((NBUF, K, D), jnp.float32),
            pltpu.SemaphoreType.DMA,
            pltpu.SemaphoreType.DMA,
            pltpu.SemaphoreType.DMA,
            pltpu.SemaphoreType.DMA,
        ],
    )
    def k(idx_hbm, table_hbm, out_hbm, idx_v, buf, gsem0, gsem1, ssem0, ssem1):
        gsems = (gsem0, gsem1)
        ssems = (ssem0, ssem1)
        wid = lax.axis_index("s") * NCORES + lax.axis_index("c")
        base = wid * CH_PER_W

        pltpu.sync_copy(idx_hbm.at[pl.ds(base, CH_PER_W)], idx_v)

        def gather(slot, c_local):
            return pltpu.make_async_copy(
                table_hbm.at[idx_v.at[c_local]], buf.at[slot], gsems[slot])

        def scatter(slot, c_local):
            return pltpu.make_async_copy(
                buf.at[slot], out_hbm.at[wid, pl.ds(c_local * K, K)],
                ssems[slot])

        for b in range(NBUF):
            gather(b, b).start()

        def step(g, carry):
            for b in range(NBUF):
                c = g * NBUF + b
                gather(b, c).wait()
                scatter(b, c).start()
                scatter(b, c).wait()

                @pl.when(c + NBUF < CH_PER_W)
                def _():
                    gather(b, c + NBUF).start()

            return carry

        lax.fori_loop(0, CH_PER_W // NBUF, step, 0)

    return k(idx2d, table)


def kernel(prefix, embedding):
    idx2d = prefix.reshape(NCH, K)
    return _sc_gather(idx2d, embedding)
